# baseline (speedup 1.0000x reference)
import functools

import jax
import jax.numpy as jnp
from jax import lax
from jax.experimental import pallas as pl
from jax.experimental.pallas import tpu as pltpu
from jax.experimental.pallas import tpu_sc as plsc

N = 10000
E = 320000
D = 128
G = 64

NPAD = 10240
NC, NS, L = 2, 16, 16
CHUNK = 128
NCHUNK = 80
EPAD = NC * NS * NCHUNK * CHUNK
ROWS_PER_S = NPAD // NS
BLK = 256
NBLK = NPAD // BLK

TOTAL_CHUNKS = EPAD // CHUNK
FCHUNK = 160
SCHUNK = TOTAL_CHUNKS // NS - FCHUNK
FAST_CORE = 1
IDXBUF = 56


def _sc_mesh():
    return plsc.VectorSubcoreMesh(core_axis_name="c", subcore_axis_name="s")


@functools.partial(
    pl.kernel,
    out_type=jax.ShapeDtypeStruct((NC, NPAD, D), jnp.float32),
    mesh=_sc_mesh(),
    scratch_types=[
        pltpu.VMEM((NCHUNK, CHUNK), jnp.int32),
        pltpu.VMEM((CHUNK, D), jnp.float32),
        pltpu.VMEM_SHARED((NPAD, D), jnp.float32),
    ],
)
def _sc_degree(dstc, onesD, zerosD, out, dst_v, ones_v, acc):
    c = lax.axis_index("c")
    s = lax.axis_index("s")
    pltpu.sync_copy(zerosD, ones_v)
    for i in range(ROWS_PER_S // CHUNK):
        r0 = s * ROWS_PER_S + i * CHUNK
        pltpu.sync_copy(ones_v, acc.at[pl.ds(r0, CHUNK)])
    pltpu.sync_copy(onesD, ones_v)
    plsc.subcore_barrier()
    pltpu.sync_copy(dstc.at[c, s], dst_v)

    def step(j, carry):
        pltpu.sync_copy(ones_v, acc.at[dst_v.at[j]], add=True)
        return carry

    lax.fori_loop(0, NCHUNK, step, 0)
    plsc.subcore_barrier()
    for i in range(ROWS_PER_S // CHUNK):
        r0 = s * ROWS_PER_S + i * CHUNK
        pltpu.sync_copy(acc.at[pl.ds(r0, CHUNK)], ones_v)
        pltpu.sync_copy(ones_v, out.at[c, pl.ds(r0, CHUNK)])


@functools.partial(
    pl.kernel,
    out_type=jax.ShapeDtypeStruct((NC, NPAD, D), jnp.float32),
    mesh=_sc_mesh(),
    scratch_types=[
        pltpu.VMEM((IDXBUF, CHUNK), jnp.int32),
        pltpu.VMEM((IDXBUF, CHUNK), jnp.int32),
        pltpu.VMEM((2, CHUNK, D), jnp.float32),
        pltpu.SemaphoreType.DMA,
        pltpu.VMEM_SHARED((NPAD, D), jnp.float32),
    ],
)
def _sc_scatter(y, srcc, dstc, zerosD, out, src_v, dst_v, rows_v, sem, acc):
    c = lax.axis_index("c")
    s = lax.axis_index("s")
    pltpu.sync_copy(zerosD, rows_v.at[0])
    for i in range(ROWS_PER_S // CHUNK):
        pltpu.sync_copy(rows_v.at[0], acc.at[pl.ds(s * ROWS_PER_S + i * CHUNK, CHUNK)])
    plsc.subcore_barrier()

    def run_edges(base, nchunk):
        done = 0
        while done < nchunk:
            n = min(IDXBUF, nchunk - done)
            b = base + done
            pltpu.sync_copy(srcc.at[pl.ds(b, n)], src_v.at[pl.ds(0, n)])
            pltpu.sync_copy(dstc.at[pl.ds(b, n)], dst_v.at[pl.ds(0, n)])
            pltpu.make_async_copy(y.at[src_v.at[0]], rows_v.at[0], sem).start()

            def step(j, carry):
                p = lax.rem(j, 2)
                pltpu.make_async_copy(y.at[src_v.at[j]], rows_v.at[p], sem).wait()

                @pl.when(j < n - 1)
                def _prefetch():
                    pltpu.make_async_copy(y.at[src_v.at[j + 1]],
                                          rows_v.at[1 - p], sem).start()

                pltpu.sync_copy(rows_v.at[p], acc.at[dst_v.at[j]], add=True)
                return carry

            lax.fori_loop(0, n, step, 0)
            done += n

    @pl.when(c == FAST_CORE)
    def _fast():
        run_edges(s * FCHUNK, FCHUNK)

    @pl.when(c != FAST_CORE)
    def _slow():
        run_edges(NS * FCHUNK + s * SCHUNK, SCHUNK)

    plsc.subcore_barrier()
    for i in range(ROWS_PER_S // CHUNK):
        r0 = s * ROWS_PER_S + i * CHUNK
        pltpu.sync_copy(acc.at[pl.ds(r0, CHUNK)], rows_v.at[0])
        pltpu.sync_copy(rows_v.at[0], out.at[c, pl.ds(r0, CHUNK)])


def _dis_block(degp):
    deg = degp[0, :, 0] + degp[1, :, 0] + 1.0
    return lax.rsqrt(deg)


def _tc_scale_matmul_kernel(x_ref, w_ref, degp_ref, y_ref):
    dis = _dis_block(degp_ref[...])
    y_ref[...] = dis[:, None] * jnp.dot(x_ref[...], w_ref[...],
                                        preferred_element_type=jnp.float32)


def _tc_scale_matmul(x_p, w, degp):
    return pl.pallas_call(
        _tc_scale_matmul_kernel,
        grid=(NBLK,),
        in_specs=[
            pl.BlockSpec((BLK, D), lambda i: (i, 0)),
            pl.BlockSpec((D, D), lambda i: (0, 0)),
            pl.BlockSpec((NC, BLK, D), lambda i: (0, i, 0)),
        ],
        out_specs=pl.BlockSpec((BLK, D), lambda i: (i, 0)),
        out_shape=jax.ShapeDtypeStruct((NPAD, D), jnp.float32),
    )(x_p, w, degp)


def _tc_combine_matmul_kernel(sp_ref, y_ref, degp_ref, b_ref, w_ref, out_ref):
    dis = _dis_block(degp_ref[...])
    sp = sp_ref[...]
    h = dis[:, None] * (sp[0] + sp[1] + y_ref[...]) + b_ref[...]
    h = jnp.maximum(h, 0.0)
    out_ref[...] = dis[:, None] * jnp.dot(h, w_ref[...],
                                          preferred_element_type=jnp.float32)


def _tc_combine_matmul(sp, y, degp, b2d, w):
    return pl.pallas_call(
        _tc_combine_matmul_kernel,
        grid=(NBLK,),
        in_specs=[
            pl.BlockSpec((NC, BLK, D), lambda i: (0, i, 0)),
            pl.BlockSpec((BLK, D), lambda i: (i, 0)),
            pl.BlockSpec((NC, BLK, D), lambda i: (0, i, 0)),
            pl.BlockSpec((1, D), lambda i: (0, 0)),
            pl.BlockSpec((D, D), lambda i: (0, 0)),
        ],
        out_specs=pl.BlockSpec((BLK, D), lambda i: (i, 0)),
        out_shape=jax.ShapeDtypeStruct((NPAD, D), jnp.float32),
    )(sp, y, degp, b2d, w)


def _tc_pool_kernel(sp_ref, y_ref, degp_ref, b_ref, batch_ref, wl_ref, bl_ref,
                    out_ref, gsum, cnt):
    i = pl.program_id(0)

    @pl.when(i == 0)
    def _init():
        gsum[...] = jnp.zeros((G, D), jnp.float32)
        cnt[...] = jnp.zeros((G, 128), jnp.float32)

    dis = _dis_block(degp_ref[...])
    sp = sp_ref[...]
    h = dis[:, None] * (sp[0] + sp[1] + y_ref[...]) + b_ref[...]
    h = jnp.maximum(h, 0.0)
    b = batch_ref[0]
    gid = lax.broadcasted_iota(jnp.int32, (G, BLK), 0)
    onehot = (jnp.broadcast_to(b, (G, BLK)) == gid).astype(jnp.float32)
    gsum[...] += jnp.dot(onehot, h, preferred_element_type=jnp.float32)
    cnt[...] += jnp.broadcast_to(jnp.sum(onehot, axis=1, keepdims=True), (G, 128))

    @pl.when(i == NBLK - 1)
    def _fin():
        g = gsum[...] / jnp.maximum(cnt[...], 1.0)
        out_ref[...] = jnp.dot(g, wl_ref[...],
                               preferred_element_type=jnp.float32) + bl_ref[...]


def _tc_pool(sp, y, degp, b2d, batch3, wlin, blin2d):
    return pl.pallas_call(
        _tc_pool_kernel,
        grid=(NBLK,),
        in_specs=[
            pl.BlockSpec((NC, BLK, D), lambda i: (0, i, 0)),
            pl.BlockSpec((BLK, D), lambda i: (i, 0)),
            pl.BlockSpec((NC, BLK, D), lambda i: (0, i, 0)),
            pl.BlockSpec((1, D), lambda i: (0, 0)),
            pl.BlockSpec((1, 1, BLK), lambda i: (i, 0, 0)),
            pl.BlockSpec((D, 2), lambda i: (0, 0)),
            pl.BlockSpec((1, 2), lambda i: (0, 0)),
        ],
        out_specs=pl.BlockSpec((G, 2), lambda i: (0, 0)),
        out_shape=jax.ShapeDtypeStruct((G, 2), jnp.float32),
        scratch_shapes=[
            pltpu.VMEM((G, D), jnp.float32),
            pltpu.VMEM((G, 128), jnp.float32),
        ],
    )(sp, y, degp, b2d, batch3, wlin, blin2d)


def kernel(x, edge_index, batch, W1, b1, W2, b2, Wlin, blin):
    src = edge_index[0].astype(jnp.int32)
    dst = edge_index[1].astype(jnp.int32)
    pad = jnp.full((EPAD - E,), N, jnp.int32)
    src_flat = jnp.concatenate([src, pad]).reshape(TOTAL_CHUNKS, CHUNK)
    dst_flat = jnp.concatenate([dst, pad]).reshape(TOTAL_CHUNKS, CHUNK)
    dstc = dst_flat.reshape(NC, NS, NCHUNK, CHUNK)

    x_p = jnp.zeros((NPAD, D), x.dtype).at[:N].set(x)
    batch3 = (jnp.full((NPAD,), G, jnp.int32).at[:N]
              .set(batch.astype(jnp.int32)).reshape(NBLK, 1, BLK))

    onesD = jnp.ones((CHUNK, D), jnp.float32)
    zerosD = jnp.zeros((CHUNK, D), jnp.float32)
    b1_2d = b1.reshape(1, D)
    b2_2d = b2.reshape(1, D)
    blin2d = blin.reshape(1, 2)

    degp = _sc_degree(dstc, onesD, zerosD)
    y1 = _tc_scale_matmul(x_p, W1, degp)
    s1 = _sc_scatter(y1, src_flat, dst_flat, zerosD)
    y2 = _tc_combine_matmul(s1, y1, degp, b1_2d, W2)
    s2 = _sc_scatter(y2, src_flat, dst_flat, zerosD)
    return _tc_pool(s2, y2, degp, b2_2d, batch3, Wlin, blin2d)


# speedup vs baseline: 1.3809x; 1.3809x over previous
import functools

import jax
import jax.numpy as jnp
from jax import lax
from jax.experimental import pallas as pl
from jax.experimental.pallas import tpu as pltpu
from jax.experimental.pallas import tpu_sc as plsc

N = 10000
E = 320000
D = 128
G = 64

NPAD = 10240
NC, NS, L = 2, 16, 16
CHUNK = 128
NCHUNK = 80
EPAD = NC * NS * NCHUNK * CHUNK
ROWS_PER_S = NPAD // NS
BLK = 256
NBLK = NPAD // BLK

TOTAL_CHUNKS = EPAD // CHUNK
FCHUNK = 144
SCHUNK = TOTAL_CHUNKS // NS - FCHUNK
FAST_CORE = 1
IDXBUF = 56


def _sc_mesh():
    return plsc.VectorSubcoreMesh(core_axis_name="c", subcore_axis_name="s")


@functools.partial(
    pl.kernel,
    out_type=jax.ShapeDtypeStruct((NC, NPAD, D), jnp.float32),
    mesh=_sc_mesh(),
    scratch_types=[
        pltpu.VMEM((NCHUNK, CHUNK), jnp.int32),
        pltpu.VMEM((CHUNK, D), jnp.float32),
        pltpu.VMEM_SHARED((NPAD, D), jnp.float32),
    ],
)
def _sc_degree(dstc, onesD, zerosD, out, dst_v, ones_v, acc):
    c = lax.axis_index("c")
    s = lax.axis_index("s")
    pltpu.sync_copy(zerosD, ones_v)
    for i in range(ROWS_PER_S // CHUNK):
        r0 = s * ROWS_PER_S + i * CHUNK
        pltpu.sync_copy(ones_v, acc.at[pl.ds(r0, CHUNK)])
    pltpu.sync_copy(onesD, ones_v)
    plsc.subcore_barrier()
    pltpu.sync_copy(dstc.at[c, s], dst_v)

    def step(j, carry):
        pltpu.sync_copy(ones_v, acc.at[dst_v.at[j]], add=True)
        return carry

    lax.fori_loop(0, NCHUNK, step, 0)
    plsc.subcore_barrier()
    for i in range(ROWS_PER_S // CHUNK):
        r0 = s * ROWS_PER_S + i * CHUNK
        pltpu.sync_copy(acc.at[pl.ds(r0, CHUNK)], ones_v)
        pltpu.sync_copy(ones_v, out.at[c, pl.ds(r0, CHUNK)])


@functools.partial(
    pl.kernel,
    out_type=jax.ShapeDtypeStruct((NC, NPAD, D), jnp.float32),
    mesh=_sc_mesh(),
    scratch_types=[
        pltpu.VMEM((IDXBUF, CHUNK), jnp.int32),
        pltpu.VMEM((IDXBUF, CHUNK), jnp.int32),
        pltpu.VMEM((2, CHUNK, D), jnp.float32),
        pltpu.SemaphoreType.DMA,
        pltpu.VMEM_SHARED((NPAD, D), jnp.float32),
    ],
)
def _sc_scatter(y, srcc, dstc, zerosD, out, src_v, dst_v, rows_v, sem, acc):
    c = lax.axis_index("c")
    s = lax.axis_index("s")
    pltpu.sync_copy(zerosD, rows_v.at[0])
    for i in range(ROWS_PER_S // CHUNK):
        pltpu.sync_copy(rows_v.at[0], acc.at[pl.ds(s * ROWS_PER_S + i * CHUNK, CHUNK)])
    plsc.subcore_barrier()

    def run_edges(base, nchunk):
        done = 0
        while done < nchunk:
            n = min(IDXBUF, nchunk - done)
            b = base + done
            pltpu.sync_copy(srcc.at[pl.ds(b, n)], src_v.at[pl.ds(0, n)])
            pltpu.sync_copy(dstc.at[pl.ds(b, n)], dst_v.at[pl.ds(0, n)])
            pltpu.make_async_copy(y.at[src_v.at[0]], rows_v.at[0], sem).start()

            def step(j, carry):
                p = lax.rem(j, 2)
                pltpu.make_async_copy(y.at[src_v.at[j]], rows_v.at[p], sem).wait()

                @pl.when(j < n - 1)
                def _prefetch():
                    pltpu.make_async_copy(y.at[src_v.at[j + 1]],
                                          rows_v.at[1 - p], sem).start()

                pltpu.sync_copy(rows_v.at[p], acc.at[dst_v.at[j]], add=True)
                return carry

            lax.fori_loop(0, n, step, 0)
            done += n

    @pl.when(c == FAST_CORE)
    def _fast():
        run_edges(s * FCHUNK, FCHUNK)

    @pl.when(c != FAST_CORE)
    def _slow():
        run_edges(NS * FCHUNK + s * SCHUNK, SCHUNK)

    plsc.subcore_barrier()
    for i in range(ROWS_PER_S // CHUNK):
        r0 = s * ROWS_PER_S + i * CHUNK
        pltpu.sync_copy(acc.at[pl.ds(r0, CHUNK)], rows_v.at[0])
        pltpu.sync_copy(rows_v.at[0], out.at[c, pl.ds(r0, CHUNK)])


def _dis_block(degp):
    deg = degp[0, :, 0] + degp[1, :, 0] + 1.0
    return lax.rsqrt(deg)


def _tc_scale_matmul_kernel(x_ref, w_ref, degp_ref, y_ref):
    dis = _dis_block(degp_ref[...])
    y_ref[...] = dis[:, None] * jnp.dot(x_ref[...], w_ref[...],
                                        preferred_element_type=jnp.float32)


def _tc_scale_matmul(x_p, w, degp):
    return pl.pallas_call(
        _tc_scale_matmul_kernel,
        grid=(NBLK,),
        in_specs=[
            pl.BlockSpec((BLK, D), lambda i: (i, 0)),
            pl.BlockSpec((D, D), lambda i: (0, 0)),
            pl.BlockSpec((NC, BLK, D), lambda i: (0, i, 0)),
        ],
        out_specs=pl.BlockSpec((BLK, D), lambda i: (i, 0)),
        out_shape=jax.ShapeDtypeStruct((NPAD, D), jnp.float32),
    )(x_p, w, degp)


def _tc_combine_matmul_kernel(sp_ref, y_ref, degp_ref, b_ref, w_ref, out_ref):
    dis = _dis_block(degp_ref[...])
    sp = sp_ref[...]
    h = dis[:, None] * (sp[0] + sp[1] + y_ref[...]) + b_ref[...]
    h = jnp.maximum(h, 0.0)
    out_ref[...] = dis[:, None] * jnp.dot(h, w_ref[...],
                                          preferred_element_type=jnp.float32)


def _tc_combine_matmul(sp, y, degp, b2d, w):
    return pl.pallas_call(
        _tc_combine_matmul_kernel,
        grid=(NBLK,),
        in_specs=[
            pl.BlockSpec((NC, BLK, D), lambda i: (0, i, 0)),
            pl.BlockSpec((BLK, D), lambda i: (i, 0)),
            pl.BlockSpec((NC, BLK, D), lambda i: (0, i, 0)),
            pl.BlockSpec((1, D), lambda i: (0, 0)),
            pl.BlockSpec((D, D), lambda i: (0, 0)),
        ],
        out_specs=pl.BlockSpec((BLK, D), lambda i: (i, 0)),
        out_shape=jax.ShapeDtypeStruct((NPAD, D), jnp.float32),
    )(sp, y, degp, b2d, w)


def _tc_pool_kernel(sp_ref, y_ref, degp_ref, b_ref, batch_ref, wl_ref, bl_ref,
                    out_ref, gsum, cnt):
    i = pl.program_id(0)

    @pl.when(i == 0)
    def _init():
        gsum[...] = jnp.zeros((G, D), jnp.float32)
        cnt[...] = jnp.zeros((G, 128), jnp.float32)

    dis = _dis_block(degp_ref[...])
    sp = sp_ref[...]
    h = dis[:, None] * (sp[0] + sp[1] + y_ref[...]) + b_ref[...]
    h = jnp.maximum(h, 0.0)
    b = batch_ref[0]
    gid = lax.broadcasted_iota(jnp.int32, (G, BLK), 0)
    onehot = (jnp.broadcast_to(b, (G, BLK)) == gid).astype(jnp.float32)
    gsum[...] += jnp.dot(onehot, h, preferred_element_type=jnp.float32)
    cnt[...] += jnp.broadcast_to(jnp.sum(onehot, axis=1, keepdims=True), (G, 128))

    @pl.when(i == NBLK - 1)
    def _fin():
        g = gsum[...] / jnp.maximum(cnt[...], 1.0)
        out_ref[...] = jnp.dot(g, wl_ref[...],
                               preferred_element_type=jnp.float32) + bl_ref[...]


def _tc_pool(sp, y, degp, b2d, batch3, wlin, blin2d):
    return pl.pallas_call(
        _tc_pool_kernel,
        grid=(NBLK,),
        in_specs=[
            pl.BlockSpec((NC, BLK, D), lambda i: (0, i, 0)),
            pl.BlockSpec((BLK, D), lambda i: (i, 0)),
            pl.BlockSpec((NC, BLK, D), lambda i: (0, i, 0)),
            pl.BlockSpec((1, D), lambda i: (0, 0)),
            pl.BlockSpec((1, 1, BLK), lambda i: (i, 0, 0)),
            pl.BlockSpec((D, 2), lambda i: (0, 0)),
            pl.BlockSpec((1, 2), lambda i: (0, 0)),
        ],
        out_specs=pl.BlockSpec((G, 2), lambda i: (0, 0)),
        out_shape=jax.ShapeDtypeStruct((G, 2), jnp.float32),
        scratch_shapes=[
            pltpu.VMEM((G, D), jnp.float32),
            pltpu.VMEM((G, 128), jnp.float32),
        ],
    )(sp, y, degp, b2d, batch3, wlin, blin2d)


def kernel(x, edge_index, batch, W1, b1, W2, b2, Wlin, blin):
    src = edge_index[0].astype(jnp.int32)
    dst = edge_index[1].astype(jnp.int32)
    pad = jnp.full((EPAD - E,), N, jnp.int32)
    src_flat = jnp.concatenate([src, pad]).reshape(TOTAL_CHUNKS, CHUNK)
    dst_flat = jnp.concatenate([dst, pad]).reshape(TOTAL_CHUNKS, CHUNK)
    dstc = dst_flat.reshape(NC, NS, NCHUNK, CHUNK)

    x_p = jnp.zeros((NPAD, D), x.dtype).at[:N].set(x)
    batch3 = (jnp.full((NPAD,), G, jnp.int32).at[:N]
              .set(batch.astype(jnp.int32)).reshape(NBLK, 1, BLK))

    onesD = jnp.ones((CHUNK, D), jnp.float32)
    zerosD = jnp.zeros((CHUNK, D), jnp.float32)
    b1_2d = b1.reshape(1, D)
    b2_2d = b2.reshape(1, D)
    blin2d = blin.reshape(1, 2)

    degp = _sc_degree(dstc, onesD, zerosD)
    y1 = _tc_scale_matmul(x_p, W1, degp)
    s1 = _sc_scatter(y1, src_flat, dst_flat, zerosD)
    y2 = _tc_combine_matmul(s1, y1, degp, b1_2d, W2)
    s2 = _sc_scatter(y2, src_flat, dst_flat, zerosD)
    return _tc_pool(s2, y2, degp, b2_2d, batch3, Wlin, blin2d)
